# baseline (speedup 1.0000x reference)
import functools

import jax
import jax.numpy as jnp
from jax import lax
from jax.experimental import pallas as pl
from jax.experimental.pallas import tpu as pltpu
from jax.experimental.pallas import tpu_sc as plsc

B = 4096
F = 32
NC = 2
NS = 16
NW = NC * NS
BPW = B // NW
L = 16
GROUPS = BPW // L


def _sc_gather_dot(user_ids, item_ids, user_factors, item_factors,
                   user_bias, item_bias):
  mesh = plsc.VectorSubcoreMesh(
      core_axis_name="c", subcore_axis_name="s",
      num_cores=NC, num_subcores=NS)

  @functools.partial(
      pl.kernel,
      out_type=(
          jax.ShapeDtypeStruct((B,), jnp.float32),
          jax.ShapeDtypeStruct((B,), jnp.float32),
          jax.ShapeDtypeStruct((B,), jnp.float32),
      ),
      mesh=mesh,
      compiler_params=pltpu.CompilerParams(
          needs_layout_passes=False, use_tc_tiling_on_sc=False),
      scratch_types=[
          pltpu.VMEM((BPW,), jnp.int32),
          pltpu.VMEM((BPW,), jnp.int32),
          pltpu.VMEM((BPW, F), jnp.float32),
          pltpu.VMEM((BPW, F), jnp.float32),
          pltpu.VMEM((BPW,), jnp.float32),
          pltpu.VMEM((BPW,), jnp.float32),
          pltpu.VMEM((BPW,), jnp.float32),
          pltpu.SemaphoreType.DMA,
          pltpu.SemaphoreType.DMA,
          pltpu.SemaphoreType.DMA,
          pltpu.SemaphoreType.DMA,
      ],
  )
  def sc_kernel(uid_hbm, iid_hbm, uf_hbm, itf_hbm, ubias_hbm, ibias_hbm,
                ub_out, ib_out, dot_out,
                uidx_v, iidx_v, uf_v, itf_v, ub_v, ib_v, dot_v,
                sem0, sem1, sem2, sem3):
    wid = lax.axis_index("s") * NC + lax.axis_index("c")
    base = wid * BPW

    pltpu.sync_copy(uid_hbm.at[pl.ds(base, BPW)], uidx_v)
    pltpu.sync_copy(iid_hbm.at[pl.ds(base, BPW)], iidx_v)

    c0 = pltpu.async_copy(uf_hbm.at[uidx_v], uf_v, sem0)
    c1 = pltpu.async_copy(itf_hbm.at[iidx_v], itf_v, sem1)
    c2 = pltpu.async_copy(ubias_hbm.at[uidx_v], ub_v, sem2)
    c3 = pltpu.async_copy(ibias_hbm.at[iidx_v], ib_v, sem3)
    c2.wait()
    pltpu.sync_copy(ub_v, ub_out.at[pl.ds(base, BPW)])
    c3.wait()
    pltpu.sync_copy(ib_v, ib_out.at[pl.ds(base, BPW)])
    c0.wait()
    c1.wait()

    def group_body(g, carry):
      row_idx = g * L + lax.iota(jnp.int32, 16)
      acc = jnp.zeros((16,), jnp.float32)
      for k in range(F):
        col = jnp.full((16,), k, jnp.int32)
        u = plsc.load_gather(uf_v, [row_idx, col])
        it = plsc.load_gather(itf_v, [row_idx, col])
        acc = acc + u * it
      dot_v[pl.ds(g * L, L)] = acc
      return carry

    lax.fori_loop(0, GROUPS, group_body, 0)
    pltpu.sync_copy(dot_v, dot_out.at[pl.ds(base, BPW)])

  return sc_kernel(user_ids, item_ids, user_factors, item_factors,
                   user_bias, item_bias)


def _tc_broadcast_body(ub_ref, ib_ref, dot_ref, out_ref):
  out_ref[...] = (ub_ref[...] + ib_ref[...]) + dot_ref[...]


def kernel(user_ids, item_ids, user_factors, item_factors, user_bias,
           item_bias):
  ub, ib, dot = _sc_gather_dot(
      user_ids, item_ids, user_factors, item_factors,
      user_bias.reshape(-1), item_bias.reshape(-1))
  ub = ub.reshape(B, 1)
  ib = ib.reshape(B, 1)
  dot2d = dot.reshape(1, B)

  BM = 256
  out = pl.pallas_call(
      _tc_broadcast_body,
      grid=(B // BM,),
      in_specs=[
          pl.BlockSpec((BM, 1), lambda i: (i, 0)),
          pl.BlockSpec((BM, 1), lambda i: (i, 0)),
          pl.BlockSpec((1, B), lambda i: (0, 0)),
      ],
      out_specs=pl.BlockSpec((BM, B), lambda i: (i, 0)),
      out_shape=jax.ShapeDtypeStruct((B, B), jnp.float32),
      compiler_params=pltpu.CompilerParams(
          dimension_semantics=("arbitrary",)),
  )(ub, ib, dot2d)
  return out


# speedup vs baseline: 6.1220x; 6.1220x over previous
import functools

import jax
import jax.numpy as jnp
from jax import lax
from jax.experimental import pallas as pl
from jax.experimental.pallas import tpu as pltpu
from jax.experimental.pallas import tpu_sc as plsc

B = 4096
F = 32
NC = 2
NS = 16
NW = NC * NS
BPW = B // NW
L = 16
GROUPS = BPW // L


def _sc_bias_sum(user_ids, item_ids, user_bias, item_bias):
  mesh = plsc.VectorSubcoreMesh(
      core_axis_name="c", subcore_axis_name="s",
      num_cores=NC, num_subcores=NS)

  @functools.partial(
      pl.kernel,
      out_type=jax.ShapeDtypeStruct((B,), jnp.float32),
      mesh=mesh,
      compiler_params=pltpu.CompilerParams(
          needs_layout_passes=False, use_tc_tiling_on_sc=False),
      scratch_types=[
          pltpu.VMEM((BPW,), jnp.int32),
          pltpu.VMEM((BPW,), jnp.int32),
          pltpu.VMEM((BPW,), jnp.float32),
          pltpu.VMEM((BPW,), jnp.float32),
          pltpu.VMEM((BPW,), jnp.float32),
          pltpu.SemaphoreType.DMA,
          pltpu.SemaphoreType.DMA,
      ],
  )
  def sc_kernel(uid_hbm, iid_hbm, ubias_hbm, ibias_hbm, s_out,
                uidx_v, iidx_v, ub_v, ib_v, s_v, sem0, sem1):
    wid = lax.axis_index("s") * NC + lax.axis_index("c")
    base = wid * BPW

    pltpu.sync_copy(uid_hbm.at[pl.ds(base, BPW)], uidx_v)
    pltpu.sync_copy(iid_hbm.at[pl.ds(base, BPW)], iidx_v)

    c0 = pltpu.async_copy(ubias_hbm.at[uidx_v], ub_v, sem0)
    c1 = pltpu.async_copy(ibias_hbm.at[iidx_v], ib_v, sem1)
    c0.wait()
    c1.wait()

    def group_body(g, carry):
      sl = pl.ds(g * L, L)
      s_v[sl] = ub_v[sl] + ib_v[sl]
      return carry

    lax.fori_loop(0, GROUPS, group_body, 0)
    pltpu.sync_copy(s_v, s_out.at[pl.ds(base, BPW)])

  return sc_kernel(user_ids, item_ids, user_bias, item_bias)


def _tc_dot_broadcast_body(ufT_ref, itT_ref, s_ref, out_ref):
  prod = ufT_ref[...] * itT_ref[...]
  dotrow = jnp.sum(prod, axis=0, keepdims=True)
  out_ref[...] = s_ref[...] + dotrow


def kernel(user_ids, item_ids, user_factors, item_factors, user_bias,
           item_bias):
  ufg = jnp.take(user_factors, user_ids, axis=0, mode="clip")
  itg = jnp.take(item_factors, item_ids, axis=0, mode="clip")
  s = _sc_bias_sum(user_ids, item_ids,
                   user_bias.reshape(-1), item_bias.reshape(-1))
  s2d = s.reshape(B, 1)

  BM = 512
  out = pl.pallas_call(
      _tc_dot_broadcast_body,
      grid=(B // BM,),
      in_specs=[
          pl.BlockSpec((F, B), lambda i: (0, 0)),
          pl.BlockSpec((F, B), lambda i: (0, 0)),
          pl.BlockSpec((BM, 1), lambda i: (i, 0)),
      ],
      out_specs=pl.BlockSpec((BM, B), lambda i: (i, 0)),
      out_shape=jax.ShapeDtypeStruct((B, B), jnp.float32),
      compiler_params=pltpu.CompilerParams(
          dimension_semantics=("parallel",)),
  )(ufg.T, itg.T, s2d)
  return out


# speedup vs baseline: 12.1161x; 1.9791x over previous
import functools

import jax
import jax.numpy as jnp
from jax import lax
from jax.experimental import pallas as pl
from jax.experimental.pallas import tpu as pltpu
from jax.experimental.pallas import tpu_sc as plsc

B = 4096
F = 32
NC = 2
NS = 16
NW = NC * NS
BPW = B // NW
L = 16
GROUPS = BPW // L


def _sc_bias_sum(ub_g, ib_g):
  mesh = plsc.VectorSubcoreMesh(
      core_axis_name="c", subcore_axis_name="s",
      num_cores=NC, num_subcores=NS)

  @functools.partial(
      pl.kernel,
      out_type=jax.ShapeDtypeStruct((B,), jnp.float32),
      mesh=mesh,
      compiler_params=pltpu.CompilerParams(
          needs_layout_passes=False, use_tc_tiling_on_sc=False),
      scratch_types=[
          pltpu.VMEM((BPW,), jnp.float32),
          pltpu.VMEM((BPW,), jnp.float32),
          pltpu.VMEM((BPW,), jnp.float32),
      ],
  )
  def sc_kernel(ub_hbm, ib_hbm, s_out, ub_v, ib_v, s_v):
    wid = lax.axis_index("s") * NC + lax.axis_index("c")
    base = wid * BPW

    pltpu.sync_copy(ub_hbm.at[pl.ds(base, BPW)], ub_v)
    pltpu.sync_copy(ib_hbm.at[pl.ds(base, BPW)], ib_v)

    def group_body(g, carry):
      sl = pl.ds(g * L, L)
      s_v[sl] = ub_v[sl] + ib_v[sl]
      return carry

    lax.fori_loop(0, GROUPS, group_body, 0)
    pltpu.sync_copy(s_v, s_out.at[pl.ds(base, BPW)])

  return sc_kernel(ub_g, ib_g)


def _tc_dot_broadcast_body(ufT_ref, itT_ref, s_ref, out_ref):
  prod = ufT_ref[...] * itT_ref[...]
  dotrow = jnp.sum(prod, axis=0, keepdims=True)
  out_ref[...] = s_ref[...] + dotrow


def kernel(user_ids, item_ids, user_factors, item_factors, user_bias,
           item_bias):
  ufg = jnp.take(user_factors, user_ids, axis=0, mode="clip")
  itg = jnp.take(item_factors, item_ids, axis=0, mode="clip")
  zeros = jnp.zeros_like(user_ids)
  ub_g = user_bias[user_ids, zeros]
  ib_g = item_bias[item_ids, zeros]
  s = _sc_bias_sum(ub_g, ib_g)
  s2d = s.reshape(B, 1)

  BM = 512
  out = pl.pallas_call(
      _tc_dot_broadcast_body,
      grid=(B // BM,),
      in_specs=[
          pl.BlockSpec((F, B), lambda i: (0, 0)),
          pl.BlockSpec((F, B), lambda i: (0, 0)),
          pl.BlockSpec((BM, 1), lambda i: (i, 0)),
      ],
      out_specs=pl.BlockSpec((BM, B), lambda i: (i, 0)),
      out_shape=jax.ShapeDtypeStruct((B, B), jnp.float32),
      compiler_params=pltpu.CompilerParams(
          dimension_semantics=("parallel",)),
  )(ufg.T, itg.T, s2d)
  return out
